# baseline (speedup 1.0000x reference)
import functools

import jax
import jax.numpy as jnp
from jax import lax
from jax.experimental import pallas as pl
from jax.experimental.pallas import tpu as pltpu
from jax.experimental.pallas import tpu_sc as plsc

N_NODES = 10000
D = 128
N_EDGES = 320000
NC = 2
NS = 16
NW = NC * NS
L = 16
CH = 64
NBUF = 4
NIB = 8
NCHUNK = 160
EPT = NCHUNK * CH
E_PAD = NW * EPT
P = 10112
RPT = P // NS


def _sc_scatter(x, e3, zeros2, zeros1):
    mesh = plsc.VectorSubcoreMesh(core_axis_name="c", subcore_axis_name="s")

    @functools.partial(
        pl.kernel,
        mesh=mesh,
        out_type=[
            jax.ShapeDtypeStruct((NC, P, D), jnp.float32),
            jax.ShapeDtypeStruct((NW, P), jnp.float32),
        ],
        scratch_types=[
            pltpu.VMEM_SHARED((P, D), jnp.float32),
            pltpu.VMEM((P,), jnp.float32),
        ] + [pltpu.VMEM((2, CH), jnp.int32)] * NIB
          + [pltpu.VMEM((CH, D), jnp.float32)] * NBUF
          + [pltpu.SemaphoreType.DMA] * (NIB + 2 * NBUF),
        compiler_params=pltpu.CompilerParams(needs_layout_passes=False),
    )
    def k(x_hbm, e_hbm, z2_hbm, z1_hbm, psum_hbm, degs_hbm,
          acc, degb, *bufs):
        idxbs = bufs[:NIB]
        rowbs = bufs[NIB:NIB + NBUF]
        isems = bufs[NIB + NBUF:2 * NIB + NBUF]
        gsems = bufs[2 * NIB + NBUF:2 * NIB + NBUF + NBUF]
        ssems = bufs[2 * NIB + NBUF + NBUF:]
        c = lax.axis_index("c")
        s = lax.axis_index("s")
        wid = s * NC + c

        for i in range(NIB):
            pltpu.async_copy(e_hbm.at[wid, i], idxbs[i], isems[i])
        for b in range(2):
            pltpu.make_async_copy(e_hbm.at[wid, b], idxbs[b], isems[b]).wait()
            pltpu.async_copy(x_hbm.at[idxbs[b].at[0]], rowbs[b], gsems[b])
        pltpu.sync_copy(z2_hbm.at[pl.ds(s * RPT, RPT)],
                        acc.at[pl.ds(s * RPT, RPT)])
        pltpu.sync_copy(z1_hbm, degb)
        plsc.subcore_barrier()

        ones = jnp.full((L,), 1.0, jnp.float32)

        def group(go, carry):
            for i in range(NIB):
                g = go * NIB + i
                b = i % NBUF
                pltpu.make_async_copy(
                    x_hbm.at[pl.ds(0, CH)], rowbs[b], gsems[b]).wait()
                pltpu.async_copy(
                    rowbs[b], acc.at[idxbs[i].at[1]], ssems[b], add=True)
                for j in range(CH // L):
                    idx = idxbs[i][1, pl.ds(j * L, L)]
                    plsc.addupdate_scatter(degb, [idx], ones)
                b2 = (i + 2) % NBUF
                s2 = (i + 2) % NIB
                s6 = (i + 6) % NIB

                @pl.when(g >= 2)
                def _wait_scat():
                    pltpu.make_async_copy(
                        rowbs[b2], acc.at[idxbs[s6].at[1]], ssems[b2]).wait()

                @pl.when(g < NCHUNK - 2)
                def _fire_gather():
                    pltpu.make_async_copy(
                        e_hbm.at[wid, 0], idxbs[s2], isems[s2]).wait()
                    pltpu.async_copy(
                        x_hbm.at[idxbs[s2].at[0]], rowbs[b2], gsems[b2])

                @pl.when(jnp.logical_and(g >= 2, g <= NCHUNK - 7))
                def _refill_idx():
                    pltpu.async_copy(e_hbm.at[wid, g + 6], idxbs[s6], isems[s6])
            return carry

        lax.fori_loop(0, NCHUNK // NIB, group, 0)
        for g in (NCHUNK - 2, NCHUNK - 1):
            pltpu.make_async_copy(
                rowbs[g % NBUF], acc.at[idxbs[g % NIB].at[1]],
                ssems[g % NBUF]).wait()
        plsc.subcore_barrier()
        pltpu.sync_copy(acc.at[pl.ds(s * RPT, RPT)],
                        psum_hbm.at[c, pl.ds(s * RPT, RPT)])
        pltpu.sync_copy(degb, degs_hbm.at[wid])

    return k(x, e3, zeros2, zeros1)


BR = 128


def _combine(psum, degs):
    def body(p_ref, d_ref, o_ref):
        p = p_ref[...]
        d = jnp.sum(d_ref[...], axis=0)
        o_ref[...] = (p[0] + p[1]) / jnp.maximum(d, 1.0)[:, None]

    return pl.pallas_call(
        body,
        grid=(P // BR,),
        in_specs=[
            pl.BlockSpec((NC, BR, D), lambda i: (0, i, 0)),
            pl.BlockSpec((NW, BR), lambda i: (0, i)),
        ],
        out_specs=pl.BlockSpec((BR, D), lambda i: (i, 0)),
        out_shape=jax.ShapeDtypeStruct((P, D), jnp.float32),
    )(psum, degs)


def kernel(x, edge_index):
    ei = edge_index.astype(jnp.int32)
    pad = E_PAD - N_EDGES
    src = jnp.pad(ei[0], (0, pad)).reshape(NW, NCHUNK, 1, CH)
    dst = jnp.pad(ei[1], (0, pad), constant_values=N_NODES).reshape(NW, NCHUNK, 1, CH)
    e3 = jnp.concatenate([src, dst], axis=2)
    zeros2 = jnp.zeros((P, D), jnp.float32)
    zeros1 = jnp.zeros((P,), jnp.float32)
    psum, degs = _sc_scatter(x, e3, zeros2, zeros1)
    return _combine(psum, degs)[:N_NODES]


# speedup vs baseline: 2.5505x; 2.5505x over previous
import functools

import jax
import jax.numpy as jnp
from jax import lax
from jax.experimental import pallas as pl
from jax.experimental.pallas import tpu as pltpu
from jax.experimental.pallas import tpu_sc as plsc

N_NODES = 10000
D = 128
DH = D // 2
N_EDGES = 320000
NC = 2
NS = 16
L = 16
CH = 128
NBUF = 4
NIB = 8
NCHUNK = 160
EPT = NCHUNK * CH
E_PAD = NS * EPT
P = 10112
RPT = P // NS
XS = 632
XL = N_NODES - 15 * XS


def _sc_scatter(x, e3, zeros2, zeros1):
    mesh = plsc.VectorSubcoreMesh(core_axis_name="c", subcore_axis_name="s")

    @functools.partial(
        pl.kernel,
        mesh=mesh,
        out_type=[
            jax.ShapeDtypeStruct((NC, P, DH), jnp.float32),
            jax.ShapeDtypeStruct((NS, P), jnp.float32),
        ],
        scratch_types=[
            pltpu.VMEM_SHARED((P, DH), jnp.float32),
            pltpu.VMEM_SHARED((N_NODES, DH), jnp.float32),
            pltpu.VMEM((P,), jnp.float32),
        ] + [pltpu.VMEM((2, CH), jnp.int32)] * NIB
          + [pltpu.VMEM((CH, DH), jnp.float32)] * NBUF
          + [pltpu.SemaphoreType.DMA] * (NIB + 2 * NBUF),
        compiler_params=pltpu.CompilerParams(
            needs_layout_passes=False, use_tc_tiling_on_sc=False),
    )
    def k(x_hbm, e_hbm, z2_hbm, z1_hbm, psum_hbm, degs_hbm,
          acc, xsp, degb, *bufs):
        idxbs = bufs[:NIB]
        rowbs = bufs[NIB:NIB + NBUF]
        isems = bufs[NIB + NBUF:2 * NIB + NBUF]
        gsems = bufs[2 * NIB + NBUF:2 * NIB + NBUF + NBUF]
        ssems = bufs[2 * NIB + NBUF + NBUF:]
        c = lax.axis_index("c")
        s = lax.axis_index("s")

        for i in range(NIB):
            pltpu.async_copy(e_hbm.at[s, i], idxbs[i], isems[i])
        @pl.when(s < NS - 1)
        def _stage():
            pltpu.sync_copy(
                x_hbm.at[c, pl.ds(s * XS, XS)],
                xsp.at[pl.ds(s * XS, XS)])

        @pl.when(s == NS - 1)
        def _stage_last():
            pltpu.sync_copy(
                x_hbm.at[c, pl.ds((NS - 1) * XS, XL)],
                xsp.at[pl.ds((NS - 1) * XS, XL)])

        pltpu.sync_copy(z2_hbm.at[pl.ds(s * RPT, RPT)],
                        acc.at[pl.ds(s * RPT, RPT)])

        @pl.when(c == 0)
        def _zero_hist():
            pltpu.sync_copy(z1_hbm, degb)

        plsc.subcore_barrier()
        for b in range(2):
            pltpu.make_async_copy(e_hbm.at[s, b], idxbs[b], isems[b]).wait()
            pltpu.async_copy(xsp.at[idxbs[b].at[0]], rowbs[b], gsems[b])

        ones = jnp.full((L,), 1.0, jnp.float32)

        def group(go, carry):
            for i in range(NIB):
                g = go * NIB + i
                b = i % NBUF
                pltpu.make_async_copy(
                    xsp.at[pl.ds(0, CH)], rowbs[b], gsems[b]).wait()
                pltpu.async_copy(
                    rowbs[b], acc.at[idxbs[i].at[1]], ssems[b], add=True)

                @pl.when(c == 0)
                def _degrees():
                    for j in range(CH // L):
                        idx = idxbs[i][1, pl.ds(j * L, L)]
                        plsc.addupdate_scatter(degb, [idx], ones)

                b2 = (i + 2) % NBUF
                s2 = (i + 2) % NIB
                s6 = (i + 6) % NIB

                @pl.when(g >= 2)
                def _wait_scat():
                    pltpu.make_async_copy(
                        rowbs[b2], acc.at[idxbs[s6].at[1]], ssems[b2]).wait()

                @pl.when(g < NCHUNK - 2)
                def _fire_gather():
                    pltpu.make_async_copy(
                        e_hbm.at[s, 0], idxbs[s2], isems[s2]).wait()
                    pltpu.async_copy(
                        xsp.at[idxbs[s2].at[0]], rowbs[b2], gsems[b2])

                @pl.when(jnp.logical_and(g >= 2, g <= NCHUNK - 7))
                def _refill_idx():
                    pltpu.async_copy(e_hbm.at[s, g + 6], idxbs[s6], isems[s6])
            return carry

        lax.fori_loop(0, NCHUNK // NIB, group, 0)
        for g in (NCHUNK - 2, NCHUNK - 1):
            pltpu.make_async_copy(
                rowbs[g % NBUF], acc.at[idxbs[g % NIB].at[1]],
                ssems[g % NBUF]).wait()
        plsc.subcore_barrier()
        pltpu.sync_copy(acc.at[pl.ds(s * RPT, RPT)],
                        psum_hbm.at[c, pl.ds(s * RPT, RPT)])

        @pl.when(c == 0)
        def _export_hist():
            pltpu.sync_copy(degb, degs_hbm.at[s])

    return k(x, e3, zeros2, zeros1)


BR = 128


def _combine(psum, degs):
    def body(p_ref, d_ref, o_ref):
        p = p_ref[...]
        d = jnp.sum(d_ref[...], axis=0)
        o_ref[...] = jnp.concatenate([p[0], p[1]], axis=1) / jnp.maximum(
            d, 1.0)[:, None]

    return pl.pallas_call(
        body,
        grid=(P // BR,),
        in_specs=[
            pl.BlockSpec((NC, BR, DH), lambda i: (0, i, 0)),
            pl.BlockSpec((NS, BR), lambda i: (0, i)),
        ],
        out_specs=pl.BlockSpec((BR, D), lambda i: (i, 0)),
        out_shape=jax.ShapeDtypeStruct((P, D), jnp.float32),
    )(psum, degs)


def kernel(x, edge_index):
    x2 = jnp.stack([x[:, :DH], x[:, DH:]])
    ei = edge_index.astype(jnp.int32)
    pad = E_PAD - N_EDGES
    src = jnp.pad(ei[0], (0, pad)).reshape(NS, NCHUNK, 1, CH)
    dst = jnp.pad(ei[1], (0, pad), constant_values=N_NODES).reshape(NS, NCHUNK, 1, CH)
    e3 = jnp.concatenate([src, dst], axis=2)
    zeros2 = jnp.zeros((P, DH), jnp.float32)
    zeros1 = jnp.zeros((P,), jnp.float32)
    psum, degs = _sc_scatter(x2, e3, zeros2, zeros1)
    return _combine(psum, degs)[:N_NODES]


# speedup vs baseline: 2.7213x; 1.0670x over previous
import functools

import jax
import jax.numpy as jnp
from jax import lax
from jax.experimental import pallas as pl
from jax.experimental.pallas import tpu as pltpu
from jax.experimental.pallas import tpu_sc as plsc

N_NODES = 10000
D = 128
DH = D // 2
N_EDGES = 320000
NC = 2
NS = 16
L = 16
CH = 128
NBUF = 4
NIB = 8
NCHUNK = 160
EPT = NCHUNK * CH
E_PAD = NS * EPT
P = 10112
RPT = P // NS
XS = 632
XL = N_NODES - 15 * XS


def _sc_scatter(x, e3, zeros2, zeros1):
    mesh = plsc.VectorSubcoreMesh(core_axis_name="c", subcore_axis_name="s")

    @functools.partial(
        pl.kernel,
        mesh=mesh,
        out_type=[
            jax.ShapeDtypeStruct((NC, P, DH), jnp.float32),
            jax.ShapeDtypeStruct((NS, P), jnp.float32),
        ],
        scratch_types=[
            pltpu.VMEM_SHARED((P, DH), jnp.float32),
            pltpu.VMEM_SHARED((N_NODES, DH), jnp.float32),
            pltpu.VMEM((P,), jnp.float32),
        ] + [pltpu.VMEM((2, CH), jnp.int32)] * NIB
          + [pltpu.VMEM((CH, DH), jnp.float32)] * NBUF
          + [pltpu.SemaphoreType.DMA] * (NIB + 2 * NBUF),
        compiler_params=pltpu.CompilerParams(
            needs_layout_passes=False, use_tc_tiling_on_sc=False),
    )
    def k(x_hbm, e_hbm, z2_hbm, z1_hbm, psum_hbm, degs_hbm,
          acc, xsp, degb, *bufs):
        idxbs = bufs[:NIB]
        rowbs = bufs[NIB:NIB + NBUF]
        isems = bufs[NIB + NBUF:2 * NIB + NBUF]
        gsems = bufs[2 * NIB + NBUF:2 * NIB + NBUF + NBUF]
        ssems = bufs[2 * NIB + NBUF + NBUF:]
        c = lax.axis_index("c")
        s = lax.axis_index("s")

        for i in range(NIB):
            pltpu.async_copy(e_hbm.at[s, i], idxbs[i], isems[i])
        @pl.when(s < NS - 1)
        def _stage():
            pltpu.sync_copy(
                x_hbm.at[pl.ds(s * XS, XS), pl.ds(c * DH, DH)],
                xsp.at[pl.ds(s * XS, XS)])

        @pl.when(s == NS - 1)
        def _stage_last():
            pltpu.sync_copy(
                x_hbm.at[pl.ds((NS - 1) * XS, XL), pl.ds(c * DH, DH)],
                xsp.at[pl.ds((NS - 1) * XS, XL)])

        pltpu.sync_copy(z2_hbm.at[pl.ds(s * RPT, RPT)],
                        acc.at[pl.ds(s * RPT, RPT)])

        @pl.when(c == 0)
        def _zero_hist():
            pltpu.sync_copy(z1_hbm, degb)

        plsc.subcore_barrier()
        for b in range(2):
            pltpu.make_async_copy(e_hbm.at[s, b], idxbs[b], isems[b]).wait()
            pltpu.async_copy(xsp.at[idxbs[b].at[0]], rowbs[b], gsems[b])

        ones = jnp.full((L,), 1.0, jnp.float32)

        def group(go, carry):
            for i in range(NIB):
                g = go * NIB + i
                b = i % NBUF
                pltpu.make_async_copy(
                    xsp.at[pl.ds(0, CH)], rowbs[b], gsems[b]).wait()
                pltpu.async_copy(
                    rowbs[b], acc.at[idxbs[i].at[1]], ssems[b], add=True)

                @pl.when(c == 0)
                def _degrees():
                    for j in range(CH // L):
                        idx = idxbs[i][1, pl.ds(j * L, L)]
                        plsc.addupdate_scatter(degb, [idx], ones)

                b2 = (i + 2) % NBUF
                s2 = (i + 2) % NIB
                s6 = (i + 6) % NIB

                @pl.when(g >= 2)
                def _wait_scat():
                    pltpu.make_async_copy(
                        rowbs[b2], acc.at[idxbs[s6].at[1]], ssems[b2]).wait()

                @pl.when(g < NCHUNK - 2)
                def _fire_gather():
                    pltpu.make_async_copy(
                        e_hbm.at[s, 0], idxbs[s2], isems[s2]).wait()
                    pltpu.async_copy(
                        xsp.at[idxbs[s2].at[0]], rowbs[b2], gsems[b2])

                @pl.when(jnp.logical_and(g >= 2, g <= NCHUNK - 7))
                def _refill_idx():
                    pltpu.async_copy(e_hbm.at[s, g + 6], idxbs[s6], isems[s6])
            return carry

        lax.fori_loop(0, NCHUNK // NIB, group, 0)
        for g in (NCHUNK - 2, NCHUNK - 1):
            pltpu.make_async_copy(
                rowbs[g % NBUF], acc.at[idxbs[g % NIB].at[1]],
                ssems[g % NBUF]).wait()
        plsc.subcore_barrier()
        pltpu.sync_copy(acc.at[pl.ds(s * RPT, RPT)],
                        psum_hbm.at[c, pl.ds(s * RPT, RPT)])

        @pl.when(c == 0)
        def _export_hist():
            pltpu.sync_copy(degb, degs_hbm.at[s])

    return k(x, e3, zeros2, zeros1)


BR = 128


def _combine(psum, degs):
    def body(p_ref, d_ref, o_ref):
        p = p_ref[...]
        d = jnp.sum(d_ref[...], axis=0)
        o_ref[...] = jnp.concatenate([p[0], p[1]], axis=1) / jnp.maximum(
            d, 1.0)[:, None]

    return pl.pallas_call(
        body,
        grid=(P // BR,),
        in_specs=[
            pl.BlockSpec((NC, BR, DH), lambda i: (0, i, 0)),
            pl.BlockSpec((NS, BR), lambda i: (0, i)),
        ],
        out_specs=pl.BlockSpec((BR, D), lambda i: (i, 0)),
        out_shape=jax.ShapeDtypeStruct((P, D), jnp.float32),
    )(psum, degs)


def kernel(x, edge_index):
    ei = edge_index.astype(jnp.int32)
    pad = E_PAD - N_EDGES
    src = jnp.pad(ei[0], (0, pad)).reshape(NS, NCHUNK, 1, CH)
    dst = jnp.pad(ei[1], (0, pad), constant_values=N_NODES).reshape(NS, NCHUNK, 1, CH)
    e3 = jnp.concatenate([src, dst], axis=2)
    zeros2 = jnp.zeros((P, DH), jnp.float32)
    zeros1 = jnp.zeros((P,), jnp.float32)
    psum, degs = _sc_scatter(x, e3, zeros2, zeros1)
    return _combine(psum, degs)[:N_NODES]
